# baseline (speedup 1.0000x reference)
import jax
import jax.numpy as jnp
from jax import lax
from jax.experimental import pallas as pl
from jax.experimental.pallas import tpu as pltpu


def _vq_body(x_ref, w_ref, dist_hbm, p_ref, q_ref, xout_ref,
             dscr, dsem):
    i = pl.program_id(0)
    nsteps = pl.num_programs(0)
    r = x_ref.shape[0]

    x = x_ref[...]
    w = w_ref[...]
    x2 = jnp.sum(x * x, axis=1, keepdims=True)
    w2 = jnp.sum(w * w, axis=1)
    xw = lax.dot_general(x, w, (((1,), (1,)), ((), ())),
                         preferred_element_type=jnp.float32)
    dist = x2 + w2[None, :] - 2.0 * xw

    @pl.when(i >= 1)
    def _drain_prev():
        pltpu.make_async_copy(dscr, dist_hbm.at[pl.ds((i - 1) * r, r), :],
                              dsem).wait()

    dscr[...] = dist
    pltpu.make_async_copy(dscr, dist_hbm.at[pl.ds(i * r, r), :],
                          dsem).start()

    neg = -dist
    m = jnp.max(neg, axis=1, keepdims=True)
    e = jnp.exp(neg - m)
    s_inv = 1.0 / jnp.sum(e, axis=1, keepdims=True)
    p_ref[...] = e * s_inv
    ew = lax.dot_general(e, w, (((1,), (0,)), ((), ())),
                         preferred_element_type=jnp.float32)
    q_ref[...] = ew * s_inv
    xout_ref[...] = x

    @pl.when(i == nsteps - 1)
    def _drain_last():
        pltpu.make_async_copy(dscr, dist_hbm.at[pl.ds(i * r, r), :],
                              dsem).wait()


def kernel(feat, vq_weight):
    b, c, h, w = feat.shape
    k = vq_weight.shape[0]
    n = b * h * w
    r_tile = 256
    nr = n // r_tile
    flat = jnp.transpose(feat, (0, 2, 3, 1)).reshape(n, c)

    dist, p_flat, q, x_out = pl.pallas_call(
        _vq_body,
        grid=(nr,),
        in_specs=[
            pl.BlockSpec((r_tile, c), lambda i: (i, 0)),
            pl.BlockSpec((k, c), lambda i: (0, 0)),
        ],
        out_specs=[
            pl.BlockSpec(memory_space=pl.ANY),
            pl.BlockSpec((r_tile, k), lambda i: (i, 0)),
            pl.BlockSpec((r_tile, c), lambda i: (i, 0)),
            pl.BlockSpec((r_tile, c), lambda i: (i, 0)),
        ],
        scratch_shapes=[
            pltpu.VMEM((r_tile, k), jnp.float32),
            pltpu.SemaphoreType.DMA,
        ],
        out_shape=[
            jax.ShapeDtypeStruct((n, k), jnp.float32),
            jax.ShapeDtypeStruct((n, k), jnp.float32),
            jax.ShapeDtypeStruct((n, c), jnp.float32),
            jax.ShapeDtypeStruct((n, c), jnp.float32),
        ],
        compiler_params=pltpu.CompilerParams(
            dimension_semantics=("arbitrary",),
        ),
    )(flat, vq_weight)

    featp = x_out.reshape(b, h, w, c)
    q_feat = jnp.transpose(q.reshape(b, h, w, c), (0, 3, 1, 2))
    assignment = jnp.transpose(p_flat.reshape(b, h, w, k), (0, 3, 1, 2))
    return (featp, q_feat, assignment, dist)


# speedup vs baseline: 1.0762x; 1.0762x over previous
import jax
import jax.numpy as jnp
from jax import lax
from jax.experimental import pallas as pl
from jax.experimental.pallas import tpu as pltpu


def _vq_body(x_ref, w_ref, dist_ref, p_ref, q_ref, xout_ref):
    x = x_ref[...]
    w = w_ref[...]
    x2 = jnp.sum(x * x, axis=1, keepdims=True)
    w2 = jnp.sum(w * w, axis=1)
    xw = lax.dot_general(x, w, (((1,), (1,)), ((), ())),
                         preferred_element_type=jnp.float32)
    dist = x2 + w2[None, :] - 2.0 * xw
    dist_ref[...] = dist
    neg = -dist
    m = jnp.max(neg, axis=1, keepdims=True)
    e = jnp.exp(neg - m)
    s_inv = 1.0 / jnp.sum(e, axis=1, keepdims=True)
    p_ref[...] = e * s_inv
    ew = lax.dot_general(e, w, (((1,), (0,)), ((), ())),
                         preferred_element_type=jnp.float32)
    q_ref[...] = ew * s_inv
    xout_ref[...] = x


def kernel(feat, vq_weight):
    b, c, h, w = feat.shape
    k = vq_weight.shape[0]
    n = b * h * w
    r_tile = 256
    nr = n // r_tile
    flat = jnp.transpose(feat, (0, 2, 3, 1)).reshape(n, c)

    dist, p_flat, q, x_out = pl.pallas_call(
        _vq_body,
        grid=(nr,),
        in_specs=[
            pl.BlockSpec((r_tile, c), lambda i: (i, 0)),
            pl.BlockSpec((k, c), lambda i: (0, 0)),
        ],
        out_specs=[
            pl.BlockSpec((r_tile, k), lambda i: (i, 0)),
            pl.BlockSpec((r_tile, k), lambda i: (i, 0)),
            pl.BlockSpec((r_tile, c), lambda i: (i, 0)),
            pl.BlockSpec((r_tile, c), lambda i: (i, 0)),
        ],
        out_shape=[
            jax.ShapeDtypeStruct((n, k), jnp.float32),
            jax.ShapeDtypeStruct((n, k), jnp.float32),
            jax.ShapeDtypeStruct((n, c), jnp.float32),
            jax.ShapeDtypeStruct((n, c), jnp.float32),
        ],
        compiler_params=pltpu.CompilerParams(
            dimension_semantics=("parallel",),
        ),
    )(flat, vq_weight)

    featp = x_out.reshape(b, h, w, c)
    q_feat = jnp.transpose(q.reshape(b, h, w, c), (0, 3, 1, 2))
    assignment = jnp.transpose(p_flat.reshape(b, h, w, k), (0, 3, 1, 2))
    return (featp, q_feat, assignment, dist)


# speedup vs baseline: 1.1565x; 1.0746x over previous
import jax
import jax.numpy as jnp
from jax import lax
from jax.experimental import pallas as pl
from jax.experimental.pallas import tpu as pltpu


def _vq_body(x_ref, w_ref, dist_ref, p_ref, q_ref, xout_ref):
    x = x_ref[...]
    w = w_ref[...]
    x2 = jnp.sum(x * x, axis=1, keepdims=True)
    w2 = jnp.sum(w * w, axis=1)
    xw2 = lax.dot_general(x + x, w, (((1,), (1,)), ((), ())),
                          preferred_element_type=jnp.float32)
    dist = x2 + w2[None, :] - xw2
    dist_ref[...] = dist
    m = jnp.min(dist, axis=1, keepdims=True)
    e = jnp.exp(m - dist)
    s_inv = 1.0 / jnp.sum(e, axis=1, keepdims=True)
    p_ref[...] = e * s_inv
    ew = lax.dot_general(e, w, (((1,), (0,)), ((), ())),
                         preferred_element_type=jnp.float32)
    q_ref[...] = ew * s_inv
    xout_ref[...] = x


def kernel(feat, vq_weight):
    b, c, h, w = feat.shape
    k = vq_weight.shape[0]
    n = b * h * w
    r_tile = 256
    nr = n // r_tile
    flat = jnp.transpose(feat, (0, 2, 3, 1)).reshape(n, c)

    dist, p_flat, q, x_out = pl.pallas_call(
        _vq_body,
        grid=(nr,),
        in_specs=[
            pl.BlockSpec((r_tile, c), lambda i: (i, 0)),
            pl.BlockSpec((k, c), lambda i: (0, 0)),
        ],
        out_specs=[
            pl.BlockSpec((r_tile, k), lambda i: (i, 0)),
            pl.BlockSpec((r_tile, k), lambda i: (i, 0)),
            pl.BlockSpec((r_tile, c), lambda i: (i, 0)),
            pl.BlockSpec((r_tile, c), lambda i: (i, 0)),
        ],
        out_shape=[
            jax.ShapeDtypeStruct((n, k), jnp.float32),
            jax.ShapeDtypeStruct((n, k), jnp.float32),
            jax.ShapeDtypeStruct((n, c), jnp.float32),
            jax.ShapeDtypeStruct((n, c), jnp.float32),
        ],
        compiler_params=pltpu.CompilerParams(
            dimension_semantics=("parallel",),
        ),
    )(flat, vq_weight)

    featp = x_out.reshape(b, h, w, c)
    q_feat = jnp.transpose(q.reshape(b, h, w, c), (0, 3, 1, 2))
    assignment = jnp.transpose(p_flat.reshape(b, h, w, k), (0, 3, 1, 2))
    return (featp, q_feat, assignment, dist)
